# baseline (speedup 1.0000x reference)
import functools

import jax
import jax.numpy as jnp
import numpy as np
from jax import lax
from jax.experimental import pallas as pl
from jax.experimental.pallas import tpu as pltpu
from jax.experimental.pallas import tpu_sc as plsc

N = 10000
E = 320000
D = 128
H = 512
K = 6
TEMP = 0.5
GAMMA = 0.5

RT = 400
NT = N // RT


def _cheb_coeffs(gammas):
    j = jnp.arange(K + 1, dtype=jnp.float32)
    xj = jnp.cos((j + 0.5) * jnp.pi / (K + 1))
    kk = jnp.arange(K + 1, dtype=jnp.float32)[:, None]
    Tkx = jnp.cos(kk * jnp.arccos(jnp.clip(xj, -1.0, 1.0))[None, :])
    w = (2.0 / (K + 1)) * (Tkx @ gammas)
    w = w.at[0].set(w[0] / 2.0)
    return w


def _enc1_body(feat_ref, w1_ref, b1_ref, hpre_ref, stats_ref):
    i = pl.program_id(0)
    h = jnp.dot(feat_ref[...], w1_ref[...],
                preferred_element_type=jnp.float32) + b1_ref[...][None, :]
    hpre_ref[...] = h
    st = jnp.concatenate([jnp.sum(h, axis=0, keepdims=True),
                          jnp.sum(h * h, axis=0, keepdims=True)], axis=0)

    @pl.when(i == 0)
    def _():
        stats_ref[...] = st

    @pl.when(i != 0)
    def _():
        stats_ref[...] += st


def _enc1(feat, w1, b1):
    return pl.pallas_call(
        _enc1_body,
        grid=(NT,),
        in_specs=[
            pl.BlockSpec((RT, D), lambda i: (i, 0)),
            pl.BlockSpec((D, H), lambda i: (0, 0)),
            pl.BlockSpec((H,), lambda i: (0,)),
        ],
        out_specs=[
            pl.BlockSpec((RT, H), lambda i: (i, 0)),
            pl.BlockSpec((2, H), lambda i: (0, 0)),
        ],
        out_shape=[
            jax.ShapeDtypeStruct((N, H), jnp.float32),
            jax.ShapeDtypeStruct((2, H), jnp.float32),
        ],
    )(feat, w1, b1)


def _enc2_body(hpre_ref, stats_ref, g_ref, b_ref, w2_ref, b2_ref,
               pw_ref, pb_ref, trans_ref):
    mu = stats_ref[0, :] * (1.0 / N)
    var = stats_ref[1, :] * (1.0 / N) - mu * mu
    rstd = jax.lax.rsqrt(var + 1e-5)
    h = (hpre_ref[...] - mu[None, :]) * (rstd * g_ref[...])[None, :] \
        + b_ref[...][None, :]
    h = jnp.maximum(h, 0.0)
    tf = jnp.dot(h, w2_ref[...], preferred_element_type=jnp.float32) \
        + b2_ref[...][None, :]
    p = jnp.dot(tf, pw_ref[...], preferred_element_type=jnp.float32) \
        + pb_ref[...][None, :]
    nrm = jnp.sqrt(jnp.sum(p * p, axis=1, keepdims=True))
    trans_ref[...] = p / jnp.maximum(nrm, 1e-12)


def _enc2(hpre, stats, g, b, w2, b2, pw, pb):
    return pl.pallas_call(
        _enc2_body,
        grid=(NT,),
        in_specs=[
            pl.BlockSpec((RT, H), lambda i: (i, 0)),
            pl.BlockSpec((2, H), lambda i: (0, 0)),
            pl.BlockSpec((H,), lambda i: (0,)),
            pl.BlockSpec((H,), lambda i: (0,)),
            pl.BlockSpec((H, H), lambda i: (0, 0)),
            pl.BlockSpec((H,), lambda i: (0,)),
            pl.BlockSpec((H, H), lambda i: (0, 0)),
            pl.BlockSpec((H,), lambda i: (0,)),
        ],
        out_specs=pl.BlockSpec((RT, H), lambda i: (i, 0)),
        out_shape=jax.ShapeDtypeStruct((N, H), jnp.float32),
    )(hpre, stats, g, b, w2, b2, pw, pb)


def _cpost1_body(outs_ref, dr_ref, hs_ref, stats_ref):
    i = pl.program_id(0)
    hs = outs_ref[...] * dr_ref[...]
    hs_ref[...] = hs
    st = jnp.concatenate([jnp.sum(hs, axis=0, keepdims=True),
                          jnp.sum(hs * hs, axis=0, keepdims=True)], axis=0)

    @pl.when(i == 0)
    def _():
        stats_ref[...] = st

    @pl.when(i != 0)
    def _():
        stats_ref[...] += st


def _cpost1(outs, drecip):
    return pl.pallas_call(
        _cpost1_body,
        grid=(NT,),
        in_specs=[
            pl.BlockSpec((RT, D), lambda i: (i, 0)),
            pl.BlockSpec((RT, 1), lambda i: (i, 0)),
        ],
        out_specs=[
            pl.BlockSpec((RT, D), lambda i: (i, 0)),
            pl.BlockSpec((2, D), lambda i: (0, 0)),
        ],
        out_shape=[
            jax.ShapeDtypeStruct((N, D), jnp.float32),
            jax.ShapeDtypeStruct((2, D), jnp.float32),
        ],
    )(outs, drecip)


def _cpost2_body(hs_ref, stats_ref, g_ref, b_ref, w_ref, bb_ref, out_ref):
    mu = stats_ref[0, :] * (1.0 / N)
    var = stats_ref[1, :] * (1.0 / N) - mu * mu
    rstd = jax.lax.rsqrt(var + 1e-5)
    h = (hs_ref[...] - mu[None, :]) * (rstd * g_ref[...])[None, :] \
        + b_ref[...][None, :]
    h = jnp.dot(h, w_ref[...], preferred_element_type=jnp.float32) \
        + bb_ref[...][None, :]
    h = jnp.maximum(h, 0.0)
    nrm = jnp.sqrt(jnp.sum(h * h, axis=1, keepdims=True))
    out_ref[...] = h / jnp.maximum(nrm, 1e-12)


def _cpost2(hs, stats, g, b, w, bb):
    return pl.pallas_call(
        _cpost2_body,
        grid=(NT,),
        in_specs=[
            pl.BlockSpec((RT, D), lambda i: (i, 0)),
            pl.BlockSpec((2, D), lambda i: (0, 0)),
            pl.BlockSpec((D,), lambda i: (0,)),
            pl.BlockSpec((D,), lambda i: (0,)),
            pl.BlockSpec((D, H), lambda i: (0, 0)),
            pl.BlockSpec((H,), lambda i: (0,)),
        ],
        out_specs=pl.BlockSpec((RT, H), lambda i: (i, 0)),
        out_shape=jax.ShapeDtypeStruct((N, H), jnp.float32),
    )(hs, stats, g, b, w, bb)


def _sim_body(a_ref, b_ref, s_ref, d_ref):
    i = pl.program_id(0)
    j = pl.program_id(1)
    simT = jax.lax.dot_general(
        b_ref[...], a_ref[...], (((1,), (1,)), ((), ())),
        preferred_element_type=jnp.float32) * (1.0 / TEMP)
    e = jnp.exp(simT)
    s = jnp.sum(e, axis=0, keepdims=True)

    @pl.when(j == 0)
    def _():
        s_ref[pl.ds(i, 1), :] = s

    @pl.when(j != 0)
    def _():
        s_ref[pl.ds(i, 1), :] += s

    @pl.when(j == i)
    def _():
        rr = jax.lax.broadcasted_iota(jnp.int32, simT.shape, 0)
        cc = jax.lax.broadcasted_iota(jnp.int32, simT.shape, 1)
        dv = jnp.sum(jnp.where(rr == cc, simT, 0.0), axis=0, keepdims=True)
        d_ref[pl.ds(i, 1), :] = dv


def _sim_rowsums(a_bf, b_bf):
    return pl.pallas_call(
        _sim_body,
        grid=(NT, NT),
        in_specs=[
            pl.BlockSpec((RT, H), lambda i, j: (i, 0)),
            pl.BlockSpec((RT, H), lambda i, j: (j, 0)),
        ],
        out_specs=[
            pl.BlockSpec((NT, RT), lambda i, j: (0, 0)),
            pl.BlockSpec((NT, RT), lambda i, j: (0, 0)),
        ],
        out_shape=[
            jax.ShapeDtypeStruct((NT, RT), jnp.float32),
            jax.ShapeDtypeStruct((NT, RT), jnp.float32),
        ],
    )(a_bf, b_bf)


def _loss_body(s1_ref, dg_ref, s2_ref, es_ref, ss_ref, do_ref, out_ref):
    dg = dg_ref[...]
    l2rows = dg - jnp.log(s1_ref[...] - jnp.exp(dg))
    l3rows = ss_ref[...] / do_ref[...] - jnp.log(s2_ref[...] - es_ref[...])
    loss2 = -jnp.sum(l2rows) * (1.0 / N)
    loss3 = -jnp.sum(l3rows) * (1.0 / N)
    out_ref[...] = jnp.reshape((1.0 - GAMMA) * loss2 + GAMMA * loss3, (1, 1))


def _loss(s1, dg, s2, es, ss, do):
    full = pl.BlockSpec((NT, RT), lambda: (0, 0))
    return pl.pallas_call(
        _loss_body,
        in_specs=[full] * 6,
        out_specs=pl.BlockSpec((1, 1), lambda: (0, 0)),
        out_shape=jax.ShapeDtypeStruct((1, 1), jnp.float32),
    )(s1, dg, s2, es, ss, do)


NPAD = 10240
STRIPE = NPAD // 16
EPAD = 327680
NB128 = EPAD // 128 // 16
NB64 = EPAD // 64 // 32


def _sc_mesh():
    return plsc.VectorSubcoreMesh(core_axis_name="c", subcore_axis_name="s")


def _fill_const(ref, n, val):
    def body(i, _):
        ref[pl.ds(i * 16, 16)] = jnp.full((16,), val, jnp.float32)
        return 0
    lax.fori_loop(0, n // 16, body, 0)


def _zero_stripe(acc_row_or_2d, strb, s):
    _fill_const(strb, STRIPE, 0.0)
    pltpu.sync_copy(strb, acc_row_or_2d.at[pl.ds(s * STRIPE, STRIPE)])


def _deg_body(ed_ref, out_ref, acc, idxs, oneb, strb):
    c = lax.axis_index("c")
    s = lax.axis_index("s")
    _fill_const(oneb, 128, 1.0)
    _zero_stripe(acc, strb, s)
    pltpu.sync_copy(ed_ref.at[c].at[pl.ds(s * NB128, NB128)], idxs)
    plsc.subcore_barrier()

    def blk(b, _):
        pltpu.sync_copy(oneb, acc.at[idxs.at[b]], add=True)
        return 0

    lax.fori_loop(0, NB128, blk, 0)
    plsc.subcore_barrier()
    pltpu.sync_copy(acc.at[pl.ds(s * STRIPE, STRIPE)], strb)
    pltpu.sync_copy(strb, out_ref.at[c].at[pl.ds(s * STRIPE, STRIPE)])


def _degrees(src2d, dst2d):
    ed = jnp.stack([dst2d, src2d])
    degs = pl.kernel(
        _deg_body,
        out_type=jax.ShapeDtypeStruct((2, NPAD), jnp.float32),
        mesh=_sc_mesh(),
        scratch_types=[
            pltpu.VMEM_SHARED((NPAD,), jnp.float32),
            pltpu.VMEM((NB128, 128), jnp.int32),
            pltpu.VMEM((128,), jnp.float32),
            pltpu.VMEM((STRIPE,), jnp.float32),
        ],
    )(ed)
    return degs[0, :N], degs[1, :N]


DC = 64
CH = 128
SR = 40


def _cheb_body(xh_ref, d2_ref, src_ref, dst_ref, w1_ref, w2_ref,
               o1_ref, o2_ref, tA_ref, tB_ref,
               acc, rows0, rows1, sidxs, didxs, ebuf, pbuf, o1b, o2b,
               dbuf, w1b, w2b, gsem0, gsem1):
    c = lax.axis_index("c")
    s = lax.axis_index("s")
    pltpu.sync_copy(w1_ref, w1b)
    pltpu.sync_copy(w2_ref, w2b)

    def zero_acc():
        def zrow(r, _):
            for g in range(4):
                ebuf[r, pl.ds(g * 16, 16)] = jnp.zeros((16,), jnp.float32)
            return 0
        lax.fori_loop(0, CH, zrow, 0)
        for j in range(STRIPE // CH):
            pltpu.sync_copy(ebuf, acc.at[pl.ds(s * STRIPE + j * CH, CH)])

    def edge_phase(th_prev):
        def chunk(ci, _):
            ioff = s * NB128 + ci * SR
            pltpu.sync_copy(src_ref.at[pl.ds(ioff, SR)], sidxs)
            pltpu.sync_copy(dst_ref.at[pl.ds(ioff, SR)], didxs)
            pltpu.async_copy(th_prev.at[c].at[sidxs.at[0]], rows0, gsem0)

            def pair(p, _2):
                b0 = 2 * p
                pltpu.make_async_copy(
                    th_prev.at[c].at[sidxs.at[b0]], rows0, gsem0).wait()
                pltpu.async_copy(
                    th_prev.at[c].at[sidxs.at[b0 + 1]], rows1, gsem1)
                pltpu.sync_copy(rows0, acc.at[didxs.at[b0]], add=True)
                pltpu.make_async_copy(
                    th_prev.at[c].at[sidxs.at[b0 + 1]], rows1, gsem1).wait()

                @pl.when(p < SR // 2 - 1)
                def _():
                    pltpu.async_copy(
                        th_prev.at[c].at[sidxs.at[b0 + 2]], rows0, gsem0)

                pltpu.sync_copy(rows1, acc.at[didxs.at[b0 + 1]], add=True)
                return 0

            lax.fori_loop(0, SR // 2, pair, 0)
            return 0

        lax.fori_loop(0, NB128 // SR, chunk, 0)

    def update_phase(k, th_prev2, th_out):
        w1k = w1b[k, :]
        w2k = w2b[k, :]
        if k == 1:
            w10 = w1b[0, :]
            w20 = w2b[0, :]
        for j in range(STRIPE // CH):
            r0 = s * STRIPE + j * CH
            cds = pl.ds(r0, CH)
            pltpu.sync_copy(acc.at[cds], ebuf)
            pltpu.sync_copy(th_prev2.at[c].at[cds], pbuf)
            pltpu.sync_copy(d2_ref.at[cds], dbuf)
            if k > 1:
                pltpu.sync_copy(o1_ref.at[c].at[cds], o1b)
                pltpu.sync_copy(o2_ref.at[c].at[cds], o2b)

            def urow(r, _):
                for g in range(4):
                    sl = pl.ds(g * 16, 16)
                    dsp = dbuf[r, sl]
                    a = ebuf[r, sl]
                    pv = pbuf[r, sl]
                    if k == 1:
                        nv = dsp * a
                        o1b[r, sl] = w10 * pv + w1k * nv
                        o2b[r, sl] = w20 * pv + w2k * nv
                    else:
                        nv = 2.0 * (dsp * a) - pv
                        o1b[r, sl] = o1b[r, sl] + w1k * nv
                        o2b[r, sl] = o2b[r, sl] + w2k * nv
                    pbuf[r, sl] = nv
                return 0

            lax.fori_loop(0, CH, urow, 0)
            pltpu.sync_copy(pbuf, th_out.at[c].at[cds])
            pltpu.sync_copy(o1b, o1_ref.at[c].at[cds])
            pltpu.sync_copy(o2b, o2_ref.at[c].at[cds])

    def thbuf(k):
        if k == 0:
            return xh_ref
        return tA_ref if k % 2 == 1 else tB_ref

    for k in range(1, K + 1):
        zero_acc()
        plsc.subcore_barrier()
        edge_phase(thbuf(k - 1))
        plsc.subcore_barrier()
        update_phase(k, thbuf(max(k - 2, 0)), thbuf(k))
        plsc.subcore_barrier()


def _cheb_outs(xhat, d2, src2d, dst2d, w1v, w2v):
    sgn = jnp.array([1.0, -1.0] * 4, jnp.float32)[: K + 1]
    w1t = jnp.zeros((8, 16), jnp.float32).at[: K + 1, :].set(
        (w1v * sgn)[:, None])
    w2t = jnp.zeros((8, 16), jnp.float32).at[: K + 1, :].set(w2v[:, None])
    xh2 = jnp.stack([
        jnp.pad(xhat[:, :DC], ((0, NPAD - N), (0, 0))),
        jnp.pad(xhat[:, DC:], ((0, NPAD - N), (0, 0))),
    ])
    d2p = jnp.pad(d2, (0, NPAD - N), constant_values=1.0)
    d2x = jnp.broadcast_to(d2p[:, None], (NPAD, DC))
    shp = jax.ShapeDtypeStruct((2, NPAD, DC), jnp.float32)
    o1, o2, _, _ = pl.kernel(
        _cheb_body,
        out_type=[shp, shp, shp, shp],
        mesh=_sc_mesh(),
        compiler_params=pltpu.CompilerParams(use_tc_tiling_on_sc=False),
        scratch_types=[
            pltpu.VMEM_SHARED((NPAD, DC), jnp.float32),
            pltpu.VMEM((128, DC), jnp.float32),
            pltpu.VMEM((128, DC), jnp.float32),
            pltpu.VMEM((SR, 128), jnp.int32),
            pltpu.VMEM((SR, 128), jnp.int32),
            pltpu.VMEM((CH, DC), jnp.float32),
            pltpu.VMEM((CH, DC), jnp.float32),
            pltpu.VMEM((CH, DC), jnp.float32),
            pltpu.VMEM((CH, DC), jnp.float32),
            pltpu.VMEM((CH, DC), jnp.float32),
            pltpu.VMEM((8, 16), jnp.float32),
            pltpu.VMEM((8, 16), jnp.float32),
            pltpu.SemaphoreType.DMA,
            pltpu.SemaphoreType.DMA,
        ],
    )(xh2, d2x, src2d, dst2d, w1t, w2t)
    outs1 = jnp.concatenate([o1[0], o1[1]], axis=1)[:N]
    outs2 = jnp.concatenate([o2[0], o2[1]], axis=1)[:N]
    return outs1, outs2


def _etg_body(tp_ref, bp_ref, src_ref, dst_ref, gs_ref, gd_ref,
              sidxs, didxs, a0, b0, a1, b1, gsm0, gsm1, wsm0, wsm1):
    c = lax.axis_index("c")
    s = lax.axis_index("s")
    base = c * (NB64 * 16) + s * NB64
    pltpu.sync_copy(src_ref.at[pl.ds(base, NB64)], sidxs)
    pltpu.sync_copy(dst_ref.at[pl.ds(base, NB64)], didxs)

    def blk(b, _):
        off = (base + b) * 64
        pltpu.async_copy(tp_ref.at[sidxs.at[b]], a0, gsm0)
        pltpu.async_copy(bp_ref.at[didxs.at[b]], b0, gsm1)
        pltpu.make_async_copy(tp_ref.at[sidxs.at[b]], a0, gsm0).wait()
        pltpu.sync_copy(a0, gs_ref.at[pl.ds(off, 64)])
        pltpu.make_async_copy(bp_ref.at[didxs.at[b]], b0, gsm1).wait()
        pltpu.sync_copy(b0, gd_ref.at[pl.ds(off, 64)])
        return 0

    lax.fori_loop(0, NB64, blk, 0)


def _edot_body(gs_ref, gd_ref, v_ref, ev_ref):
    m = gs_ref[...].astype(jnp.float32) * gd_ref[...].astype(jnp.float32)
    v = jnp.sum(m, axis=1, keepdims=True) * (1.0 / TEMP)
    v_ref[...] = v
    ev_ref[...] = jnp.exp(v)


def _ets_body(v_ref, ev_ref, src_ref, out_ref,
              esacc, ssacc, sidxs, vbuf, ebuf2, strb):
    c = lax.axis_index("c")
    s = lax.axis_index("s")
    _zero_stripe(esacc, strb, s)
    _zero_stripe(ssacc, strb, s)
    base = c * (NB64 * 16) + s * NB64
    pltpu.sync_copy(src_ref.at[pl.ds(base, NB64)], sidxs)
    plsc.subcore_barrier()

    def blk(b, _):
        pltpu.sync_copy(v_ref.at[base + b], vbuf)
        pltpu.sync_copy(ev_ref.at[base + b], ebuf2)
        pltpu.sync_copy(ebuf2, esacc.at[sidxs.at[b]], add=True)
        pltpu.sync_copy(vbuf, ssacc.at[sidxs.at[b]], add=True)
        return 0

    lax.fori_loop(0, NB64, blk, 0)
    plsc.subcore_barrier()
    pltpu.sync_copy(esacc.at[pl.ds(s * STRIPE, STRIPE)], strb)
    pltpu.sync_copy(strb, out_ref.at[c].at[0].at[pl.ds(s * STRIPE, STRIPE)])
    pltpu.sync_copy(ssacc.at[pl.ds(s * STRIPE, STRIPE)], strb)
    pltpu.sync_copy(strb, out_ref.at[c].at[1].at[pl.ds(s * STRIPE, STRIPE)])


def _edge_terms(trans_bf, b2n_bf, src2e, dst2e):
    tp = jnp.pad(trans_bf, ((0, NPAD - N), (0, 0)))
    bp = jnp.pad(b2n_bf, ((0, NPAD - N), (0, 0)))
    gshp = jax.ShapeDtypeStruct((EPAD, H), jnp.bfloat16)
    rbuf = pltpu.VMEM((64, H), jnp.bfloat16)
    gs, gd = pl.kernel(
        _etg_body,
        out_type=[gshp, gshp],
        mesh=_sc_mesh(),
        compiler_params=pltpu.CompilerParams(use_tc_tiling_on_sc=False),
        scratch_types=[
            pltpu.VMEM((NB64, 64), jnp.int32),
            pltpu.VMEM((NB64, 64), jnp.int32),
            rbuf, rbuf, rbuf, rbuf,
            pltpu.SemaphoreType.DMA,
            pltpu.SemaphoreType.DMA,
            pltpu.SemaphoreType.DMA,
            pltpu.SemaphoreType.DMA,
        ],
    )(tp, bp, src2e, dst2e)

    ETB = 512
    v, ev = pl.pallas_call(
        _edot_body,
        grid=(EPAD // ETB,),
        in_specs=[
            pl.BlockSpec((ETB, H), lambda i: (i, 0)),
            pl.BlockSpec((ETB, H), lambda i: (i, 0)),
        ],
        out_specs=[
            pl.BlockSpec((ETB, 1), lambda i: (i, 0)),
            pl.BlockSpec((ETB, 1), lambda i: (i, 0)),
        ],
        out_shape=[
            jax.ShapeDtypeStruct((EPAD, 1), jnp.float32),
            jax.ShapeDtypeStruct((EPAD, 1), jnp.float32),
        ],
    )(gs, gd)

    parts = pl.kernel(
        _ets_body,
        out_type=jax.ShapeDtypeStruct((2, 2, NPAD), jnp.float32),
        mesh=_sc_mesh(),
        scratch_types=[
            pltpu.VMEM_SHARED((NPAD,), jnp.float32),
            pltpu.VMEM_SHARED((NPAD,), jnp.float32),
            pltpu.VMEM((NB64, 64), jnp.int32),
            pltpu.VMEM((64,), jnp.float32),
            pltpu.VMEM((64,), jnp.float32),
            pltpu.VMEM((STRIPE,), jnp.float32),
        ],
    )(v.reshape(EPAD // 64, 64), ev.reshape(EPAD // 64, 64), src2e)
    es = parts[0, 0, :N] + parts[1, 0, :N]
    ss = parts[0, 1, :N] + parts[1, 1, :N]
    return es, ss


def kernel(feat, edge_index, et_w1, et_b1, et_bn_g, et_bn_b, et_w2, et_b2,
           proj_w, proj_b, c1_gammas, c1_bn_g, c1_bn_b, c1_w, c1_b,
           c2_gammas, c2_bn_g, c2_bn_b, c2_w, c2_b):
    src = edge_index[0]
    dst = edge_index[1]
    epad = jnp.full((EPAD - E,), NPAD - 1, jnp.int32)
    srcp = jnp.concatenate([src, epad])
    dstp = jnp.concatenate([dst, epad])
    src2d = srcp.reshape(EPAD // 128, 128)
    dst2d = dstp.reshape(EPAD // 128, 128)
    src2e = srcp.reshape(EPAD // 64, 64)
    dst2e = dstp.reshape(EPAD // 64, 64)

    hpre, st1 = _enc1(feat, et_w1, et_b1)
    trans = _enc2(hpre, st1, et_bn_g, et_bn_b, et_w2, et_b2, proj_w, proj_b)

    deg_in, deg_out = _degrees(src2d, dst2d)
    dsafe = jnp.maximum(deg_in, 1.0)
    dinv = jax.lax.rsqrt(dsafe)
    drecip = jnp.sqrt(dsafe)
    d2 = 1.0 / dsafe
    xhat = feat * dinv[:, None]

    w1v = _cheb_coeffs(c1_gammas)
    w2v = _cheb_coeffs(c2_gammas)
    outs1, outs2 = _cheb_outs(xhat, d2, src2d, dst2d, w1v, w2v)

    hs1, cst1 = _cpost1(outs1, drecip[:, None])
    b1n = _cpost2(hs1, cst1, c1_bn_g, c1_bn_b, c1_w, c1_b)
    hs2, cst2 = _cpost1(outs2, drecip[:, None])
    b2n = _cpost2(hs2, cst2, c2_bn_g, c2_bn_b, c2_w, c2_b)

    a_bf = trans.astype(jnp.bfloat16)
    b2_bf = b2n.astype(jnp.bfloat16)
    s1, dg = _sim_rowsums(a_bf, b1n.astype(jnp.bfloat16))
    s2, _ = _sim_rowsums(a_bf, b2_bf)

    es, ss = _edge_terms(a_bf, b2_bf, src2e, dst2e)

    out = _loss(s1, dg, s2,
                es.reshape(NT, RT), ss.reshape(NT, RT),
                deg_out.reshape(NT, RT))
    return out[0, 0]


# speedup vs baseline: 1.2545x; 1.2545x over previous
import functools

import jax
import jax.numpy as jnp
import numpy as np
from jax import lax
from jax.experimental import pallas as pl
from jax.experimental.pallas import tpu as pltpu
from jax.experimental.pallas import tpu_sc as plsc

N = 10000
E = 320000
D = 128
H = 512
K = 6
TEMP = 0.5
GAMMA = 0.5

RT = 400
NT = N // RT


def _cheb_coeffs(gammas):
    j = jnp.arange(K + 1, dtype=jnp.float32)
    xj = jnp.cos((j + 0.5) * jnp.pi / (K + 1))
    kk = jnp.arange(K + 1, dtype=jnp.float32)[:, None]
    Tkx = jnp.cos(kk * jnp.arccos(jnp.clip(xj, -1.0, 1.0))[None, :])
    w = (2.0 / (K + 1)) * (Tkx @ gammas)
    w = w.at[0].set(w[0] / 2.0)
    return w


def _enc1_body(feat_ref, w1_ref, b1_ref, hpre_ref, stats_ref):
    i = pl.program_id(0)
    h = jnp.dot(feat_ref[...], w1_ref[...],
                preferred_element_type=jnp.float32) + b1_ref[...][None, :]
    hpre_ref[...] = h
    st = jnp.concatenate([jnp.sum(h, axis=0, keepdims=True),
                          jnp.sum(h * h, axis=0, keepdims=True)], axis=0)

    @pl.when(i == 0)
    def _():
        stats_ref[...] = st

    @pl.when(i != 0)
    def _():
        stats_ref[...] += st


def _enc1(feat, w1, b1):
    return pl.pallas_call(
        _enc1_body,
        grid=(NT,),
        in_specs=[
            pl.BlockSpec((RT, D), lambda i: (i, 0)),
            pl.BlockSpec((D, H), lambda i: (0, 0)),
            pl.BlockSpec((H,), lambda i: (0,)),
        ],
        out_specs=[
            pl.BlockSpec((RT, H), lambda i: (i, 0)),
            pl.BlockSpec((2, H), lambda i: (0, 0)),
        ],
        out_shape=[
            jax.ShapeDtypeStruct((N, H), jnp.float32),
            jax.ShapeDtypeStruct((2, H), jnp.float32),
        ],
    )(feat, w1, b1)


def _enc2_body(hpre_ref, stats_ref, g_ref, b_ref, w2_ref, b2_ref,
               pw_ref, pb_ref, trans_ref):
    mu = stats_ref[0, :] * (1.0 / N)
    var = stats_ref[1, :] * (1.0 / N) - mu * mu
    rstd = jax.lax.rsqrt(var + 1e-5)
    h = (hpre_ref[...] - mu[None, :]) * (rstd * g_ref[...])[None, :] \
        + b_ref[...][None, :]
    h = jnp.maximum(h, 0.0)
    tf = jnp.dot(h, w2_ref[...], preferred_element_type=jnp.float32) \
        + b2_ref[...][None, :]
    p = jnp.dot(tf, pw_ref[...], preferred_element_type=jnp.float32) \
        + pb_ref[...][None, :]
    nrm = jnp.sqrt(jnp.sum(p * p, axis=1, keepdims=True))
    trans_ref[...] = p / jnp.maximum(nrm, 1e-12)


def _enc2(hpre, stats, g, b, w2, b2, pw, pb):
    return pl.pallas_call(
        _enc2_body,
        grid=(NT,),
        in_specs=[
            pl.BlockSpec((RT, H), lambda i: (i, 0)),
            pl.BlockSpec((2, H), lambda i: (0, 0)),
            pl.BlockSpec((H,), lambda i: (0,)),
            pl.BlockSpec((H,), lambda i: (0,)),
            pl.BlockSpec((H, H), lambda i: (0, 0)),
            pl.BlockSpec((H,), lambda i: (0,)),
            pl.BlockSpec((H, H), lambda i: (0, 0)),
            pl.BlockSpec((H,), lambda i: (0,)),
        ],
        out_specs=pl.BlockSpec((RT, H), lambda i: (i, 0)),
        out_shape=jax.ShapeDtypeStruct((N, H), jnp.float32),
    )(hpre, stats, g, b, w2, b2, pw, pb)


def _cpost1_body(outs_ref, dr_ref, hs_ref, stats_ref):
    i = pl.program_id(0)
    hs = outs_ref[...] * dr_ref[...]
    hs_ref[...] = hs
    st = jnp.concatenate([jnp.sum(hs, axis=0, keepdims=True),
                          jnp.sum(hs * hs, axis=0, keepdims=True)], axis=0)

    @pl.when(i == 0)
    def _():
        stats_ref[...] = st

    @pl.when(i != 0)
    def _():
        stats_ref[...] += st


def _cpost1(outs, drecip):
    return pl.pallas_call(
        _cpost1_body,
        grid=(NT,),
        in_specs=[
            pl.BlockSpec((RT, D), lambda i: (i, 0)),
            pl.BlockSpec((RT, 1), lambda i: (i, 0)),
        ],
        out_specs=[
            pl.BlockSpec((RT, D), lambda i: (i, 0)),
            pl.BlockSpec((2, D), lambda i: (0, 0)),
        ],
        out_shape=[
            jax.ShapeDtypeStruct((N, D), jnp.float32),
            jax.ShapeDtypeStruct((2, D), jnp.float32),
        ],
    )(outs, drecip)


def _cpost2_body(hs_ref, stats_ref, g_ref, b_ref, w_ref, bb_ref, out_ref):
    mu = stats_ref[0, :] * (1.0 / N)
    var = stats_ref[1, :] * (1.0 / N) - mu * mu
    rstd = jax.lax.rsqrt(var + 1e-5)
    h = (hs_ref[...] - mu[None, :]) * (rstd * g_ref[...])[None, :] \
        + b_ref[...][None, :]
    h = jnp.dot(h, w_ref[...], preferred_element_type=jnp.float32) \
        + bb_ref[...][None, :]
    h = jnp.maximum(h, 0.0)
    nrm = jnp.sqrt(jnp.sum(h * h, axis=1, keepdims=True))
    out_ref[...] = h / jnp.maximum(nrm, 1e-12)


def _cpost2(hs, stats, g, b, w, bb):
    return pl.pallas_call(
        _cpost2_body,
        grid=(NT,),
        in_specs=[
            pl.BlockSpec((RT, D), lambda i: (i, 0)),
            pl.BlockSpec((2, D), lambda i: (0, 0)),
            pl.BlockSpec((D,), lambda i: (0,)),
            pl.BlockSpec((D,), lambda i: (0,)),
            pl.BlockSpec((D, H), lambda i: (0, 0)),
            pl.BlockSpec((H,), lambda i: (0,)),
        ],
        out_specs=pl.BlockSpec((RT, H), lambda i: (i, 0)),
        out_shape=jax.ShapeDtypeStruct((N, H), jnp.float32),
    )(hs, stats, g, b, w, bb)


def _sim_body(a_ref, b_ref, s_ref, d_ref):
    i = pl.program_id(0)
    j = pl.program_id(1)
    simT = jax.lax.dot_general(
        b_ref[...], a_ref[...], (((1,), (1,)), ((), ())),
        preferred_element_type=jnp.float32) * (1.0 / TEMP)
    e = jnp.exp(simT)
    s = jnp.sum(e, axis=0, keepdims=True)

    @pl.when(j == 0)
    def _():
        s_ref[pl.ds(i, 1), :] = s

    @pl.when(j != 0)
    def _():
        s_ref[pl.ds(i, 1), :] += s

    @pl.when(j == i)
    def _():
        rr = jax.lax.broadcasted_iota(jnp.int32, simT.shape, 0)
        cc = jax.lax.broadcasted_iota(jnp.int32, simT.shape, 1)
        dv = jnp.sum(jnp.where(rr == cc, simT, 0.0), axis=0, keepdims=True)
        d_ref[pl.ds(i, 1), :] = dv


def _sim_rowsums(a_bf, b_bf):
    return pl.pallas_call(
        _sim_body,
        grid=(NT, NT),
        in_specs=[
            pl.BlockSpec((RT, H), lambda i, j: (i, 0)),
            pl.BlockSpec((RT, H), lambda i, j: (j, 0)),
        ],
        out_specs=[
            pl.BlockSpec((NT, RT), lambda i, j: (0, 0)),
            pl.BlockSpec((NT, RT), lambda i, j: (0, 0)),
        ],
        out_shape=[
            jax.ShapeDtypeStruct((NT, RT), jnp.float32),
            jax.ShapeDtypeStruct((NT, RT), jnp.float32),
        ],
    )(a_bf, b_bf)


def _loss_body(s1_ref, dg_ref, s2_ref, es_ref, ss_ref, do_ref, out_ref):
    dg = dg_ref[...]
    l2rows = dg - jnp.log(s1_ref[...] - jnp.exp(dg))
    l3rows = ss_ref[...] / do_ref[...] - jnp.log(s2_ref[...] - es_ref[...])
    loss2 = -jnp.sum(l2rows) * (1.0 / N)
    loss3 = -jnp.sum(l3rows) * (1.0 / N)
    out_ref[...] = jnp.reshape((1.0 - GAMMA) * loss2 + GAMMA * loss3, (1, 1))


def _loss(s1, dg, s2, es, ss, do):
    full = pl.BlockSpec((NT, RT), lambda: (0, 0))
    return pl.pallas_call(
        _loss_body,
        in_specs=[full] * 6,
        out_specs=pl.BlockSpec((1, 1), lambda: (0, 0)),
        out_shape=jax.ShapeDtypeStruct((1, 1), jnp.float32),
    )(s1, dg, s2, es, ss, do)


NPAD = 10240
STRIPE = NPAD // 16
EPAD = 327680
NB128 = EPAD // 128 // 16
NB64 = EPAD // 64 // 32


def _sc_mesh():
    return plsc.VectorSubcoreMesh(core_axis_name="c", subcore_axis_name="s")


def _fill_const(ref, n, val):
    def body(i, _):
        ref[pl.ds(i * 16, 16)] = jnp.full((16,), val, jnp.float32)
        return 0
    lax.fori_loop(0, n // 16, body, 0)


def _zero_stripe(acc_row_or_2d, strb, s):
    _fill_const(strb, STRIPE, 0.0)
    pltpu.sync_copy(strb, acc_row_or_2d.at[pl.ds(s * STRIPE, STRIPE)])


def _deg_body(ed_ref, out_ref, acc, idxs, oneb, strb):
    c = lax.axis_index("c")
    s = lax.axis_index("s")
    _fill_const(oneb, 128, 1.0)
    _zero_stripe(acc, strb, s)
    pltpu.sync_copy(ed_ref.at[c].at[pl.ds(s * NB128, NB128)], idxs)
    plsc.subcore_barrier()

    def blk(b, _):
        pltpu.sync_copy(oneb, acc.at[idxs.at[b]], add=True)
        return 0

    lax.fori_loop(0, NB128, blk, 0)
    plsc.subcore_barrier()
    pltpu.sync_copy(acc.at[pl.ds(s * STRIPE, STRIPE)], strb)
    pltpu.sync_copy(strb, out_ref.at[c].at[pl.ds(s * STRIPE, STRIPE)])


def _degrees(src2d, dst2d):
    ed = jnp.stack([dst2d, src2d])
    degs = pl.kernel(
        _deg_body,
        out_type=jax.ShapeDtypeStruct((2, NPAD), jnp.float32),
        mesh=_sc_mesh(),
        scratch_types=[
            pltpu.VMEM_SHARED((NPAD,), jnp.float32),
            pltpu.VMEM((NB128, 128), jnp.int32),
            pltpu.VMEM((128,), jnp.float32),
            pltpu.VMEM((STRIPE,), jnp.float32),
        ],
    )(ed)
    return degs[0, :N], degs[1, :N]


DC = 64
CH = 128
SR = 40


def _cheb_body(xh_ref, d2_ref, src_ref, dst_ref, w1_ref, w2_ref,
               o1_ref, o2_ref, tA_ref, tB_ref,
               acc, rows0, rows1, sidxs, didxs, ebuf, pbuf, o1b, o2b,
               dbuf, w1b, w2b, gsem0, gsem1):
    c = lax.axis_index("c")
    s = lax.axis_index("s")
    pltpu.sync_copy(w1_ref, w1b)
    pltpu.sync_copy(w2_ref, w2b)

    def zero_acc():
        def zrow(r, _):
            for g in range(4):
                ebuf[r, pl.ds(g * 16, 16)] = jnp.zeros((16,), jnp.float32)
            return 0
        lax.fori_loop(0, CH, zrow, 0)
        for j in range(STRIPE // CH):
            pltpu.sync_copy(ebuf, acc.at[pl.ds(s * STRIPE + j * CH, CH)])

    def edge_phase(th_prev):
        def chunk(ci, _):
            ioff = s * NB128 + ci * SR
            pltpu.sync_copy(src_ref.at[pl.ds(ioff, SR)], sidxs)
            pltpu.sync_copy(dst_ref.at[pl.ds(ioff, SR)], didxs)
            pltpu.async_copy(th_prev.at[c].at[sidxs.at[0]], rows0, gsem0)

            def pair(p, _2):
                b0 = 2 * p
                pltpu.make_async_copy(
                    th_prev.at[c].at[sidxs.at[b0]], rows0, gsem0).wait()
                pltpu.async_copy(
                    th_prev.at[c].at[sidxs.at[b0 + 1]], rows1, gsem1)
                pltpu.sync_copy(rows0, acc.at[didxs.at[b0]], add=True)
                pltpu.make_async_copy(
                    th_prev.at[c].at[sidxs.at[b0 + 1]], rows1, gsem1).wait()

                @pl.when(p < SR // 2 - 1)
                def _():
                    pltpu.async_copy(
                        th_prev.at[c].at[sidxs.at[b0 + 2]], rows0, gsem0)

                pltpu.sync_copy(rows1, acc.at[didxs.at[b0 + 1]], add=True)
                return 0

            lax.fori_loop(0, SR // 2, pair, 0)
            return 0

        lax.fori_loop(0, NB128 // SR, chunk, 0)

    def update_phase(k, th_prev2, th_out):
        w1k = w1b[k, :]
        w2k = w2b[k, :]
        if k == 1:
            w10 = w1b[0, :]
            w20 = w2b[0, :]
        for j in range(STRIPE // CH):
            r0 = s * STRIPE + j * CH
            cds = pl.ds(r0, CH)
            pltpu.sync_copy(acc.at[cds], ebuf)
            pltpu.sync_copy(th_prev2.at[c].at[cds], pbuf)
            pltpu.sync_copy(d2_ref.at[cds], dbuf)
            if k > 1:
                pltpu.sync_copy(o1_ref.at[c].at[cds], o1b)
                pltpu.sync_copy(o2_ref.at[c].at[cds], o2b)

            def urow(r, _):
                for g in range(4):
                    sl = pl.ds(g * 16, 16)
                    dsp = dbuf[r, sl]
                    a = ebuf[r, sl]
                    pv = pbuf[r, sl]
                    if k == 1:
                        nv = dsp * a
                        o1b[r, sl] = w10 * pv + w1k * nv
                        o2b[r, sl] = w20 * pv + w2k * nv
                    else:
                        nv = 2.0 * (dsp * a) - pv
                        o1b[r, sl] = o1b[r, sl] + w1k * nv
                        o2b[r, sl] = o2b[r, sl] + w2k * nv
                    pbuf[r, sl] = nv
                return 0

            lax.fori_loop(0, CH, urow, 0)
            pltpu.sync_copy(pbuf, th_out.at[c].at[cds])
            pltpu.sync_copy(o1b, o1_ref.at[c].at[cds])
            pltpu.sync_copy(o2b, o2_ref.at[c].at[cds])

    def thbuf(k):
        if k == 0:
            return xh_ref
        return tA_ref if k % 2 == 1 else tB_ref

    for k in range(1, K + 1):
        zero_acc()
        plsc.subcore_barrier()
        edge_phase(thbuf(k - 1))
        plsc.subcore_barrier()
        update_phase(k, thbuf(max(k - 2, 0)), thbuf(k))
        plsc.subcore_barrier()


def _cheb_outs(xhat, d2, src2d, dst2d, w1v, w2v):
    sgn = jnp.array([1.0, -1.0] * 4, jnp.float32)[: K + 1]
    w1t = jnp.zeros((8, 16), jnp.float32).at[: K + 1, :].set(
        (w1v * sgn)[:, None])
    w2t = jnp.zeros((8, 16), jnp.float32).at[: K + 1, :].set(w2v[:, None])
    xh2 = jnp.stack([
        jnp.pad(xhat[:, :DC], ((0, NPAD - N), (0, 0))),
        jnp.pad(xhat[:, DC:], ((0, NPAD - N), (0, 0))),
    ])
    d2p = jnp.pad(d2, (0, NPAD - N), constant_values=1.0)
    d2x = jnp.broadcast_to(d2p[:, None], (NPAD, DC))
    shp = jax.ShapeDtypeStruct((2, NPAD, DC), jnp.float32)
    o1, o2, _, _ = pl.kernel(
        _cheb_body,
        out_type=[shp, shp, shp, shp],
        mesh=_sc_mesh(),
        compiler_params=pltpu.CompilerParams(use_tc_tiling_on_sc=False),
        scratch_types=[
            pltpu.VMEM_SHARED((NPAD, DC), jnp.float32),
            pltpu.VMEM((128, DC), jnp.float32),
            pltpu.VMEM((128, DC), jnp.float32),
            pltpu.VMEM((SR, 128), jnp.int32),
            pltpu.VMEM((SR, 128), jnp.int32),
            pltpu.VMEM((CH, DC), jnp.float32),
            pltpu.VMEM((CH, DC), jnp.float32),
            pltpu.VMEM((CH, DC), jnp.float32),
            pltpu.VMEM((CH, DC), jnp.float32),
            pltpu.VMEM((CH, DC), jnp.float32),
            pltpu.VMEM((8, 16), jnp.float32),
            pltpu.VMEM((8, 16), jnp.float32),
            pltpu.SemaphoreType.DMA,
            pltpu.SemaphoreType.DMA,
        ],
    )(xh2, d2x, src2d, dst2d, w1t, w2t)
    outs1 = jnp.concatenate([o1[0], o1[1]], axis=1)[:N]
    outs2 = jnp.concatenate([o2[0], o2[1]], axis=1)[:N]
    return outs1, outs2


def _etg_body(tp_ref, bp_ref, src_ref, dst_ref, gs_ref, gd_ref,
              sidxs, didxs, a0, b0):
    c = lax.axis_index("c")
    s = lax.axis_index("s")
    base = c * (NB64 * 16) + s * NB64
    pltpu.sync_copy(src_ref.at[pl.ds(base, NB64)], sidxs)
    pltpu.sync_copy(dst_ref.at[pl.ds(base, NB64)], didxs)

    def blk(b, _):
        off = (base + b) * 64
        pltpu.sync_copy(tp_ref.at[sidxs.at[b]], a0)
        pltpu.sync_copy(a0, gs_ref.at[pl.ds(off, 64)])
        pltpu.sync_copy(bp_ref.at[didxs.at[b]], b0)
        pltpu.sync_copy(b0, gd_ref.at[pl.ds(off, 64)])
        return 0

    lax.fori_loop(0, NB64, blk, 0)


def _edot_body(gs_ref, gd_ref, v_ref, ev_ref):
    m = gs_ref[...].astype(jnp.float32) * gd_ref[...].astype(jnp.float32)
    v = jnp.sum(m, axis=1, keepdims=True) * (1.0 / TEMP)
    v_ref[...] = v
    ev_ref[...] = jnp.exp(v)


def _ets_body(v_ref, ev_ref, src_ref, out_ref,
              esacc, ssacc, sidxs, vbuf, ebuf2, strb):
    c = lax.axis_index("c")
    s = lax.axis_index("s")
    _zero_stripe(esacc, strb, s)
    _zero_stripe(ssacc, strb, s)
    base = c * (NB64 * 16) + s * NB64
    pltpu.sync_copy(src_ref.at[pl.ds(base, NB64)], sidxs)
    plsc.subcore_barrier()

    def blk(b, _):
        pltpu.sync_copy(v_ref.at[base + b], vbuf)
        pltpu.sync_copy(ev_ref.at[base + b], ebuf2)
        pltpu.sync_copy(ebuf2, esacc.at[sidxs.at[b]], add=True)
        pltpu.sync_copy(vbuf, ssacc.at[sidxs.at[b]], add=True)
        return 0

    lax.fori_loop(0, NB64, blk, 0)
    plsc.subcore_barrier()
    pltpu.sync_copy(esacc.at[pl.ds(s * STRIPE, STRIPE)], strb)
    pltpu.sync_copy(strb, out_ref.at[c].at[0].at[pl.ds(s * STRIPE, STRIPE)])
    pltpu.sync_copy(ssacc.at[pl.ds(s * STRIPE, STRIPE)], strb)
    pltpu.sync_copy(strb, out_ref.at[c].at[1].at[pl.ds(s * STRIPE, STRIPE)])


def _edge_terms(trans, b2n, src2e, dst2e):
    tp = jnp.pad(trans, ((0, NPAD - N), (0, 0)))
    bp = jnp.pad(b2n, ((0, NPAD - N), (0, 0)))
    gshp = jax.ShapeDtypeStruct((EPAD, H), jnp.float32)
    rbuf = pltpu.VMEM((64, H), jnp.float32)
    gs, gd = pl.kernel(
        _etg_body,
        out_type=[gshp, gshp],
        mesh=_sc_mesh(),
        scratch_types=[
            pltpu.VMEM((NB64, 64), jnp.int32),
            pltpu.VMEM((NB64, 64), jnp.int32),
            rbuf, rbuf,
        ],
    )(tp, bp, src2e, dst2e)

    ETB = 512
    v, ev = pl.pallas_call(
        _edot_body,
        grid=(EPAD // ETB,),
        in_specs=[
            pl.BlockSpec((ETB, H), lambda i: (i, 0)),
            pl.BlockSpec((ETB, H), lambda i: (i, 0)),
        ],
        out_specs=[
            pl.BlockSpec((ETB, 1), lambda i: (i, 0)),
            pl.BlockSpec((ETB, 1), lambda i: (i, 0)),
        ],
        out_shape=[
            jax.ShapeDtypeStruct((EPAD, 1), jnp.float32),
            jax.ShapeDtypeStruct((EPAD, 1), jnp.float32),
        ],
    )(gs, gd)

    parts = pl.kernel(
        _ets_body,
        out_type=jax.ShapeDtypeStruct((2, 2, NPAD), jnp.float32),
        mesh=_sc_mesh(),
        scratch_types=[
            pltpu.VMEM_SHARED((NPAD,), jnp.float32),
            pltpu.VMEM_SHARED((NPAD,), jnp.float32),
            pltpu.VMEM((NB64, 64), jnp.int32),
            pltpu.VMEM((64,), jnp.float32),
            pltpu.VMEM((64,), jnp.float32),
            pltpu.VMEM((STRIPE,), jnp.float32),
        ],
    )(v.reshape(EPAD // 64, 64), ev.reshape(EPAD // 64, 64), src2e)
    es = parts[0, 0, :N] + parts[1, 0, :N]
    ss = parts[0, 1, :N] + parts[1, 1, :N]
    return es, ss


def kernel(feat, edge_index, et_w1, et_b1, et_bn_g, et_bn_b, et_w2, et_b2,
           proj_w, proj_b, c1_gammas, c1_bn_g, c1_bn_b, c1_w, c1_b,
           c2_gammas, c2_bn_g, c2_bn_b, c2_w, c2_b):
    src = edge_index[0]
    dst = edge_index[1]
    epad = jnp.full((EPAD - E,), NPAD - 1, jnp.int32)
    srcp = jnp.concatenate([src, epad])
    dstp = jnp.concatenate([dst, epad])
    src2d = srcp.reshape(EPAD // 128, 128)
    dst2d = dstp.reshape(EPAD // 128, 128)
    src2e = srcp.reshape(EPAD // 64, 64)
    dst2e = dstp.reshape(EPAD // 64, 64)

    hpre, st1 = _enc1(feat, et_w1, et_b1)
    trans = _enc2(hpre, st1, et_bn_g, et_bn_b, et_w2, et_b2, proj_w, proj_b)

    deg_in, deg_out = _degrees(src2d, dst2d)
    dsafe = jnp.maximum(deg_in, 1.0)
    dinv = jax.lax.rsqrt(dsafe)
    drecip = jnp.sqrt(dsafe)
    d2 = 1.0 / dsafe
    xhat = feat * dinv[:, None]

    w1v = _cheb_coeffs(c1_gammas)
    w2v = _cheb_coeffs(c2_gammas)
    outs1, outs2 = _cheb_outs(xhat, d2, src2d, dst2d, w1v, w2v)

    hs1, cst1 = _cpost1(outs1, drecip[:, None])
    b1n = _cpost2(hs1, cst1, c1_bn_g, c1_bn_b, c1_w, c1_b)
    hs2, cst2 = _cpost1(outs2, drecip[:, None])
    b2n = _cpost2(hs2, cst2, c2_bn_g, c2_bn_b, c2_w, c2_b)

    a_bf = trans.astype(jnp.bfloat16)
    b2_bf = b2n.astype(jnp.bfloat16)
    s1, dg = _sim_rowsums(a_bf, b1n.astype(jnp.bfloat16))
    s2, _ = _sim_rowsums(a_bf, b2_bf)

    es, ss = _edge_terms(trans, b2n, src2e, dst2e)

    out = _loss(s1, dg, s2,
                es.reshape(NT, RT), ss.reshape(NT, RT),
                deg_out.reshape(NT, RT))
    return out[0, 0]
